# baseline (speedup 1.0000x reference)
import jax
import jax.numpy as jnp
from jax import lax
from jax.experimental import pallas as pl
from jax.experimental.pallas import tpu as pltpu
from jax.experimental.pallas import tpu_sc as plsc

B = 16384
N_CAT = 26
VOCAB = 1000
EMB = 77
N_COLS = 41
NC = 2
NS = 16
NW = NC * NS
ROWS = B // NW
ROWS_H = ROWS // 2
IDXW = ROWS_H * 48
CTAB = N_CAT * VOCAB
CTMAX = CTAB - 1
CT_PER = 1664
CT_VEC = CT_PER // 16
W2 = N_COLS - 16


def _body(inp_hbm, tbl_hbm, out_hbm, inp_v, idx_v, gath_v,
          ctidx_v, ctg_v, ctab_s, sem, sem2):
    sid = lax.axis_index("s")
    wid = sid * NC + lax.axis_index("c")
    iota = lax.iota(jnp.int32, 16)
    p0 = iota * VOCAB
    l1 = iota + 16
    c1 = l1 < N_CAT
    p1 = jnp.where(c1, l1 * VOCAB, 0)
    l2 = iota + W2
    c2 = l2 < N_CAT
    p2 = jnp.where(c2, l2 * VOCAB, 0)

    a_inp = pltpu.async_copy(
        inp_hbm.reshape(NW, ROWS, N_COLS).at[wid], inp_v, sem2)

    def ct_idx(j, carry):
        e = jnp.minimum(sid * CT_PER + j * 16 + iota, CTMAX)
        ctidx_v[pl.ds(j * 16, 16)] = e * EMB
        return carry
    lax.fori_loop(0, CT_VEC, ct_idx, None)
    a_ctab = pltpu.async_copy(tbl_hbm.at[ctidx_v], ctg_v, sem)

    a_inp.wait()

    def idx_body(r0):
        def body(r, carry):
            idx_v[pl.ds((r - r0) * 48, 16)] = inp_v[r, pl.ds(0, 16)] + p0
            idx_v[pl.ds((r - r0) * 48 + 16, 16)] = inp_v[r, pl.ds(16, 16)] + p1
            idx_v[pl.ds((r - r0) * 48 + 32, 16)] = inp_v[r, pl.ds(W2, 16)] + p2
            return carry
        lax.fori_loop(r0, r0 + ROWS_H, body, None)

    out_f = inp_v.bitcast(jnp.float32)

    def merge_body(r0):
        def body(r, carry):
            v1 = inp_v[r, pl.ds(16, 16)].astype(jnp.float32)
            v2 = inp_v[r, pl.ds(W2, 16)].astype(jnp.float32)
            out_f[r, pl.ds(0, 16)] = gath_v[pl.ds((r - r0) * 48, 16)]
            out_f[r, pl.ds(16, 16)] = jnp.where(
                c1, gath_v[pl.ds((r - r0) * 48 + 16, 16)], v1)
            out_f[r, pl.ds(W2, 16)] = jnp.where(
                c2, gath_v[pl.ds((r - r0) * 48 + 32, 16)], v2)
            return carry
        lax.fori_loop(r0, r0 + ROWS_H, body, None)

    idx_body(0)
    a_ctab.wait()
    pltpu.sync_copy(ctg_v, ctab_s.at[pl.ds(sid * CT_PER, CT_PER)])
    plsc.subcore_barrier()
    pltpu.async_copy(ctab_s.at[idx_v], gath_v, sem).wait()
    merge_body(0)
    idx_body(ROWS_H)
    pltpu.async_copy(ctab_s.at[idx_v], gath_v, sem).wait()
    merge_body(ROWS_H)

    pltpu.sync_copy(out_f, out_hbm.reshape(NW, ROWS, N_COLS).at[wid])


def kernel(inputs, tables):
    mesh = plsc.VectorSubcoreMesh(core_axis_name="c", subcore_axis_name="s")
    k = pl.kernel(
        _body,
        mesh=mesh,
        out_type=jax.ShapeDtypeStruct((B, N_COLS), jnp.float32),
        scratch_types=[
            pltpu.VMEM((ROWS, N_COLS), jnp.int32),
            pltpu.VMEM((IDXW,), jnp.int32),
            pltpu.VMEM((IDXW,), jnp.float32),
            pltpu.VMEM((CT_PER,), jnp.int32),
            pltpu.VMEM((CT_PER,), jnp.float32),
            pltpu.VMEM_SHARED((NS * CT_PER,), jnp.float32),
            pltpu.SemaphoreType.DMA,
            pltpu.SemaphoreType.DMA,
        ],
    )
    return k(inputs, tables.reshape(-1))


# speedup vs baseline: 1.2265x; 1.2265x over previous
import jax
import jax.numpy as jnp
from jax import lax
from jax.experimental import pallas as pl
from jax.experimental.pallas import tpu as pltpu
from jax.experimental.pallas import tpu_sc as plsc

B = 16384
N_CAT = 26
VOCAB = 1000
EMB = 77
N_COLS = 41
NC = 2
NS = 16
NW = NC * NS
CATW = N_CAT * B // NW
NUMW = (N_COLS - N_CAT) * B // NW
NUM_BASE = N_CAT * B
CTAB = N_CAT * VOCAB
CTMAX = CTAB - 1
CT_PER = 1664
CT_VEC = CT_PER // 16
UNROLL = 4


def _body(inp_hbm, tbl_hbm, out_hbm, inp_cat_v, idx_v, gout_v,
          inp_num_v, out_num_v, ctidx_v, ctg_v, ctab_s, sem, sem2, sem3):
    sid = lax.axis_index("s")
    wid = sid * NC + lax.axis_index("c")
    cbase = wid * CATW
    nbase = NUM_BASE + wid * NUMW
    iota = lax.iota(jnp.int32, 16)

    a_cat = pltpu.async_copy(inp_hbm.at[pl.ds(cbase, CATW)], inp_cat_v, sem2)
    a_num = pltpu.async_copy(inp_hbm.at[pl.ds(nbase, NUMW)], inp_num_v, sem3)

    def ct_idx(j, carry):
        e = jnp.minimum(sid * CT_PER + j * 16 + iota, CTMAX)
        ctidx_v[pl.ds(j * 16, 16)] = e * EMB
        return carry
    lax.fori_loop(0, CT_VEC, ct_idx, None)
    a_ctab = pltpu.async_copy(tbl_hbm.at[ctidx_v], ctg_v, sem)

    a_cat.wait()

    def idx_body(k, carry):
        for u in range(UNROLL):
            off = k * (16 * UNROLL) + u * 16
            colv = ((cbase + off) >> 14) * VOCAB
            idx_v[pl.ds(off, 16)] = inp_cat_v[pl.ds(off, 16)] + colv
        return carry
    lax.fori_loop(0, CATW // (16 * UNROLL), idx_body, None)

    a_ctab.wait()
    pltpu.sync_copy(ctg_v, ctab_s.at[pl.ds(sid * CT_PER, CT_PER)])
    plsc.subcore_barrier()

    a_g = pltpu.async_copy(ctab_s.at[idx_v], gout_v, sem)

    def cast_body(k, carry):
        for u in range(UNROLL):
            off = k * (16 * UNROLL) + u * 16
            out_num_v[pl.ds(off, 16)] = (
                inp_num_v[pl.ds(off, 16)].astype(jnp.float32))
        return carry
    a_num.wait()
    lax.fori_loop(0, NUMW // (16 * UNROLL), cast_body, None)

    a_g.wait()
    pltpu.sync_copy(gout_v, out_hbm.at[pl.ds(cbase, CATW)])
    pltpu.sync_copy(out_num_v, out_hbm.at[pl.ds(nbase, NUMW)])


def kernel(inputs, tables):
    mesh = plsc.VectorSubcoreMesh(core_axis_name="c", subcore_axis_name="s")
    k = pl.kernel(
        _body,
        mesh=mesh,
        out_type=jax.ShapeDtypeStruct((N_COLS * B,), jnp.float32),
        scratch_types=[
            pltpu.VMEM((CATW,), jnp.int32),
            pltpu.VMEM((CATW,), jnp.int32),
            pltpu.VMEM((CATW,), jnp.float32),
            pltpu.VMEM((NUMW,), jnp.int32),
            pltpu.VMEM((NUMW,), jnp.float32),
            pltpu.VMEM((CT_PER,), jnp.int32),
            pltpu.VMEM((CT_PER,), jnp.float32),
            pltpu.VMEM_SHARED((NS * CT_PER,), jnp.float32),
            pltpu.SemaphoreType.DMA,
            pltpu.SemaphoreType.DMA,
            pltpu.SemaphoreType.DMA,
        ],
    )
    out_t = k(inputs.T.reshape(-1), tables.reshape(-1))
    return out_t.reshape(N_COLS, B).T
